# Initial kernel scaffold; baseline (speedup 1.0000x reference)
import jax
import jax.numpy as jnp
from jax.experimental import pallas as pl


def kernel(input, weights):
    raise NotImplementedError("write your pallas kernel here")


# speedup vs baseline: 1.2268x; 1.2268x over previous
import functools

import jax
import jax.numpy as jnp
from jax import lax
from jax.experimental import pallas as pl
from jax.experimental.pallas import tpu as pltpu
from jax.experimental.pallas import tpu_sc as plsc

B = 4
T = 8192
D = 768
PAD = 2

_info = plsc.get_sparse_core_info()
NC = _info.num_cores
NS = _info.num_subcores
NW = NC * NS
ROWS_PER_W = T // NW
CH = 64
NCHUNK = ROWS_PER_W // CH

_mesh = plsc.VectorSubcoreMesh(core_axis_name="c", subcore_axis_name="s")


@functools.partial(
    pl.kernel,
    mesh=_mesh,
    out_type=jax.ShapeDtypeStruct((B * T * D,), jnp.float32),
    scratch_types=[
        pltpu.VMEM((CH * D,), jnp.float32),
        pltpu.VMEM((CH * D,), jnp.float32),
        pltpu.SemaphoreType.DMA,
        pltpu.SemaphoreType.DMA,
        pltpu.SemaphoreType.DMA,
        pltpu.SemaphoreType.DMA,
    ],
)
def _pos_embed(w_hbm, out_hbm, buf0, buf1, ls0, ls1, ss0, ss1):
    wid = lax.axis_index("s") * NC + lax.axis_index("c")
    base = wid * ROWS_PER_W
    bufs = (buf0, buf1)
    lsems = (ls0, ls1)
    ssems = (ss0, ss1)

    def load(i):
        off = (PAD + base + i * CH) * D
        return pltpu.async_copy(w_hbm.at[pl.ds(off, CH * D)], bufs[i % 2], lsems[i % 2])

    def fire_stores(i):
        return [
            pltpu.async_copy(
                bufs[i % 2],
                out_hbm.at[pl.ds((b * T + base + i * CH) * D, CH * D)],
                ssems[i % 2],
            )
            for b in range(B)
        ]

    loads = {0: load(0), 1: load(1)}
    stores = {}
    for i in range(NCHUNK):
        loads[i].wait()
        stores[i] = fire_stores(i)
        if i + 2 < NCHUNK:
            for h in stores[i]:
                h.wait()
            loads[i + 2] = load(i + 2)
    for i in (NCHUNK - 2, NCHUNK - 1):
        for h in stores[i]:
            h.wait()


def kernel(input, weights):
    del input
    flat = _pos_embed(weights.reshape(-1))
    return flat.reshape(B, T, D)
